# baseline (speedup 1.0000x reference)
import jax
import jax.numpy as jnp
from jax.experimental import pallas as pl

_EDGE_TYPES = [("part", "pp", "part"), ("part", "pt", "torque"), ("torque", "tp", "part"),
               ("part", "pf", "force"), ("force", "fp", "part")]
_G = 50


def _gat(x_src, x_dst, ei, p, num_dst):
    h_src = x_src @ p["W_src"]
    h_dst = x_dst @ p["W_dst"]
    a_src = h_src @ p["att_src"]
    a_dst = h_dst @ p["att_dst"]
    s, d = ei[0], ei[1]
    e = jax.nn.leaky_relu(a_src[s] + a_dst[d], negative_slope=0.2)
    m = jax.ops.segment_max(e, d, num_segments=num_dst)
    m = jnp.where(jnp.isfinite(m), m, 0.0)
    ex = jnp.exp(e - m[d])
    den = jax.ops.segment_sum(ex, d, num_segments=num_dst)
    alpha = ex / (den[d] + 1e-16)
    out = jax.ops.segment_sum(alpha[:, None] * h_src[s], d, num_segments=num_dst)
    return out + p["bias"]


def _hetero(xd, edges, lp, sizes):
    new = {}
    for (st, rel, dt) in _EDGE_TYPES:
        o = _gat(xd[st], xd[dt], edges[rel], lp[rel], sizes[dt])
        new[dt] = o if dt not in new else new[dt] + o
    return new


def _aggr(x, ids, ns):
    mx = jax.ops.segment_max(x, ids, num_segments=ns)
    mn = -jax.ops.segment_max(-x, ids, num_segments=ns)
    sm = jax.ops.segment_sum(x, ids, num_segments=ns)
    cnt = jax.ops.segment_sum(jnp.ones((x.shape[0], 1), x.dtype), ids, num_segments=ns)
    return jnp.concatenate([mx, mn, sm / jnp.maximum(cnt, 1.0)], axis=1)


def _pl_identity(x):
    def body(x_ref, o_ref):
        o_ref[...] = x_ref[...]
    return pl.pallas_call(body, out_shape=jax.ShapeDtypeStruct(x.shape, x.dtype))(x)


def kernel(mass, torque_x, force_x, params, part_state, edges, part_batch, part_id, torque_batch, force_batch):
    n_part = mass.shape[0]
    n_torque = torque_x.shape[0]
    n_force = force_x.shape[0]
    sizes = {"part": n_part, "torque": n_torque, "force": n_force}
    npart_per = n_part // _G

    state_idx = part_state[:, 0] + 2 * part_state[:, 1]
    reppart = jnp.concatenate([mass @ params["ep_W"], params["state_table"][state_idx]], axis=-1)
    xd = {"part": _pl_identity(reppart), "torque": torque_x, "force": force_x}
    for li in range(4):
        xd = _hetero(xd, edges, params["convs"][li], sizes)
        if li < 3:
            xd = {k: jax.nn.relu(v) for k, v in xd.items()}
    repA = _hetero(xd, edges, params["actor"], sizes)
    h = repA["part"]
    mu = h.mean(-1, keepdims=True)
    var = h.var(-1, keepdims=True)
    rep_actions = (h - mu) / jnp.sqrt(var + 1e-5) * params["ln_w"] + params["ln_b"]
    rep_actions = rep_actions @ params["out_a_W"] + params["out_a_b"]
    m = jax.ops.segment_max(rep_actions, part_batch, num_segments=_G)
    ex = jnp.exp(rep_actions - m[part_batch])
    den = jax.ops.segment_sum(ex, part_batch, num_segments=_G)
    soft = ex / (den[part_batch] + 1e-16)
    actions = jnp.zeros((_G, 2, npart_per), jnp.float32).at[part_batch, :, part_id].set(soft)
    actions = actions.reshape(_G, -1)
    rep = jnp.concatenate([
        _aggr(xd["part"], part_batch, _G),
        _aggr(xd["torque"], torque_batch, _G),
        _aggr(xd["force"], force_batch, _G),
    ], axis=1)
    repV = jax.nn.gelu(rep @ params["innet_W"] + params["innet_b"], approximate=False)
    repV = jax.nn.gelu(repV @ params["full_W"] + params["full_b"], approximate=False)
    V = jnp.tanh(repV @ params["outnet_W"] + params["outnet_b"])
    return actions, V


# speedup vs baseline: 5.3103x; 5.3103x over previous
import functools

import jax
import jax.numpy as jnp
from jax import lax
from jax.experimental import pallas as pl
from jax.experimental.pallas import tpu as pltpu
from jax.experimental.pallas import tpu_sc as plsc

_EDGE_TYPES = [("part", "pp", "part"), ("part", "pt", "torque"), ("torque", "tp", "part"),
               ("part", "pf", "force"), ("force", "fp", "part")]
_G = 50
_HID = 64
_BUCKET = 512
_CH = 512
_NW = 32


@functools.lru_cache(maxsize=None)
def _make_conv_kernel(E_pad, NB, N_src):
    NBK = (NB + _NW - 1) // _NW
    OFF_PAD = ((NB + 1 + 15) // 16) * 16
    mesh = plsc.VectorSubcoreMesh(core_axis_name="c", subcore_axis_name="s", num_cores=2)

    @functools.partial(
        pl.kernel, mesh=mesh,
        compiler_params=pltpu.CompilerParams(needs_layout_passes=False,
                                             use_tc_tiling_on_sc=False),
        out_type=jax.ShapeDtypeStruct((NB * _BUCKET * 64,), jnp.float32),
        scratch_types=[
            pltpu.VMEM((N_src,), jnp.float32),
            pltpu.VMEM((_BUCKET,), jnp.float32),
            pltpu.VMEM((_BUCKET,), jnp.float32),
            pltpu.VMEM((_BUCKET,), jnp.float32),
            pltpu.VMEM((_BUCKET * 64,), jnp.float32),
            pltpu.VMEM((_CH,), jnp.int32),
            pltpu.VMEM((_CH,), jnp.int32),
            pltpu.VMEM((_CH, 64), jnp.float32),
            pltpu.VMEM((OFF_PAD,), jnp.int32),
            pltpu.SemaphoreType.DMA,
        ],
    )
    def conv(s_hbm, d_hbm, offs_hbm, h_hbm, asrc_hbm, adst_hbm, out_hbm,
             asrc_v, adst_v, m_v, den_v, out_v, s_v, d_v, rows_v, offs_v, sem):
        wid = lax.axis_index("s") * 2 + lax.axis_index("c")
        pltpu.sync_copy(asrc_hbm, asrc_v)
        pltpu.sync_copy(offs_hbm, offs_v)
        iota = lax.iota(jnp.int32, 16)

        def read_off(b):
            goff = (b >> 4) << 4
            grp = offs_v[pl.ds(goff, 16)]
            lane = b & 15
            return jnp.max(jnp.where(iota == lane, grp, jnp.int32(-2147483648)))

        for k in range(NBK):
            b = wid + k * _NW

            @pl.when(b < NB)
            def _process():
                base = b * _BUCKET
                start = read_off(b)
                end = read_off(b + 1)
                c0 = start >> 9
                nch = (end - (c0 << 9) + _CH - 1) >> 9
                pltpu.sync_copy(adst_hbm.at[pl.ds(base, _BUCKET)], adst_v)

                def init_small(i, _):
                    m_v[pl.ds(i * 16, 16)] = jnp.full((16,), -1e30, jnp.float32)
                    den_v[pl.ds(i * 16, 16)] = jnp.zeros((16,), jnp.float32)
                    return 0
                lax.fori_loop(0, _BUCKET // 16, init_small, 0)

                def init_out(i, _):
                    out_v[pl.ds(i * 16, 16)] = jnp.zeros((16,), jnp.float32)
                    return 0
                lax.fori_loop(0, _BUCKET * 64 // 16, init_out, 0)

                def edge_vals(i):
                    s16 = s_v[pl.ds(i * 16, 16)]
                    d16 = d_v[pl.ds(i * 16, 16)]
                    dl = d16 - base
                    mask = (dl >= 0) & (dl < _BUCKET)
                    dl = jnp.clip(dl, 0, _BUCKET - 1)
                    a_s = plsc.load_gather(asrc_v, [s16], mask=mask)
                    a_d = plsc.load_gather(adst_v, [dl], mask=mask)
                    x = a_s + a_d
                    e = jnp.where(x >= 0, x, 0.2 * x)
                    return s16, dl, mask, e

                def sweep1(c, _):
                    ceoff = (c0 + c) * _CH
                    pltpu.sync_copy(s_hbm.at[pl.ds(ceoff, _CH)], s_v)
                    pltpu.sync_copy(d_hbm.at[pl.ds(ceoff, _CH)], d_v)

                    def grp1(i, _):
                        _, dl, mask, e = edge_vals(i)
                        e = jnp.where(mask, e, -1e30)
                        cur = plsc.load_gather(m_v, [dl])
                        plsc.store_scatter(m_v, [dl], jnp.maximum(cur, e))
                        return 0
                    lax.fori_loop(0, _CH // 16, grp1, 0)
                    return 0
                lax.fori_loop(0, nch, sweep1, 0)

                def sweep2(c, _):
                    ceoff = (c0 + c) * _CH
                    pltpu.sync_copy(s_hbm.at[pl.ds(ceoff, _CH)], s_v)
                    pltpu.sync_copy(d_hbm.at[pl.ds(ceoff, _CH)], d_v)

                    def grp2(i, _):
                        _, dl, mask, e = edge_vals(i)
                        mm = plsc.load_gather(m_v, [dl])
                        ex = jnp.where(mask, jnp.exp(e - mm), 0.0)
                        plsc.addupdate_scatter(den_v, [dl], ex)
                        return 0
                    lax.fori_loop(0, _CH // 16, grp2, 0)
                    return 0
                lax.fori_loop(0, nch, sweep2, 0)

                def invert(i, _):
                    den = den_v[pl.ds(i * 16, 16)]
                    den_v[pl.ds(i * 16, 16)] = 1.0 / (den + 1e-16)
                    return 0
                lax.fori_loop(0, _BUCKET // 16, invert, 0)

                def sweep3(c, _):
                    ceoff = (c0 + c) * _CH
                    pltpu.sync_copy(s_hbm.at[pl.ds(ceoff, _CH)], s_v)
                    pltpu.sync_copy(d_hbm.at[pl.ds(ceoff, _CH)], d_v)
                    pltpu.async_copy(h_hbm.at[s_v], rows_v, sem).wait()

                    def grp3(i, _):
                        _, dl, mask, e = edge_vals(i)
                        mm = plsc.load_gather(m_v, [dl])
                        inv = plsc.load_gather(den_v, [dl])
                        ex = jnp.where(mask, jnp.exp(e - mm), 0.0)
                        alpha = ex * inv
                        eidx = i * 16 + iota
                        dlb = dl * 64
                        for t in range(64):
                            tv = jnp.full((16,), t, jnp.int32)
                            col = plsc.load_gather(rows_v, [eidx, tv])
                            plsc.addupdate_scatter(out_v, [dlb + t], col * alpha)
                        return 0
                    lax.fori_loop(0, _CH // 16, grp3, 0)
                    return 0
                lax.fori_loop(0, nch, sweep3, 0)

                pltpu.sync_copy(out_v, out_hbm.at[pl.ds(b * (_BUCKET * 64), _BUCKET * 64)])

    return conv


def _sc_gat(h_src, a_src, a_dst, binned, num_dst):
    s_b, d_b, offs, E_pad = binned
    NB = (num_dst + _BUCKET - 1) // _BUCKET
    NBP = NB * _BUCKET
    N_src = h_src.shape[0]
    N_src_pad = ((N_src + 127) // 128) * 128
    a_src = jnp.pad(a_src, (0, N_src_pad - N_src))
    conv = _make_conv_kernel(E_pad, NB, N_src_pad)
    a_dst_p = jnp.pad(a_dst, (0, NBP - num_dst))
    out = conv(s_b, d_b, offs, h_src, a_src, a_dst_p)
    return out.reshape(NBP, 64)[:num_dst]


def _bin_edges(ei, num_dst):
    s, d = ei[0], ei[1]
    E = s.shape[0]
    NB = (num_dst + _BUCKET - 1) // _BUCKET
    OFF_PAD = ((NB + 1 + 15) // 16) * 16
    E_pad = E + 1024
    perm = jnp.argsort(d)
    s_b = jnp.pad(s[perm], (0, E_pad - E))
    d_b = jnp.pad(d[perm], (0, E_pad - E), constant_values=NB * _BUCKET + 1)
    offs = jnp.searchsorted(d_b[:E], jnp.arange(NB + 1, dtype=jnp.int32) * _BUCKET,
                            side="left").astype(jnp.int32)
    offs = jnp.pad(offs, (0, OFF_PAD - NB - 1), constant_values=E)
    return s_b, d_b, offs, E_pad


def _gat_dense(x_src, x_dst, p):
    h_src = x_src @ p["W_src"]
    a_src = h_src @ p["att_src"]
    a_dst = (x_dst @ p["W_dst"]) @ p["att_dst"]
    return h_src, a_src, a_dst


def _hetero_sc(xd, binned, lp, sizes, dst_types=("part", "torque", "force")):
    new = {}
    for (st, rel, dt) in _EDGE_TYPES:
        if dt not in dst_types:
            continue
        h_src, a_src, a_dst = _gat_dense(xd[st], xd[dt], lp[rel])
        o = _sc_gat(h_src, a_src, a_dst, binned[rel], sizes[dt]) + lp[rel]["bias"]
        new[dt] = o if dt not in new else new[dt] + o
    return new


def _aggr(x, ns):
    xg = x.reshape(_G, -1, x.shape[-1])
    return jnp.concatenate([xg.max(1), xg.min(1), xg.mean(1)], axis=1)


def kernel(mass, torque_x, force_x, params, part_state, edges, part_batch, part_id, torque_batch, force_batch):
    n_part = mass.shape[0]
    n_torque = torque_x.shape[0]
    n_force = force_x.shape[0]
    sizes = {"part": n_part, "torque": n_torque, "force": n_force}
    npart_per = n_part // _G

    binned = {rel: _bin_edges(edges[rel], sizes[dt]) for (st, rel, dt) in _EDGE_TYPES}

    state_idx = part_state[:, 0] + 2 * part_state[:, 1]
    reppart = jnp.concatenate([mass @ params["ep_W"], params["state_table"][state_idx]], axis=-1)
    xd = {"part": reppart, "torque": torque_x, "force": force_x}
    for li in range(4):
        xd = _hetero_sc(xd, binned, params["convs"][li], sizes)
        if li < 3:
            xd = {k: jax.nn.relu(v) for k, v in xd.items()}
    repA = _hetero_sc(xd, binned, params["actor"], sizes, dst_types=("part",))
    h = repA["part"]
    mu = h.mean(-1, keepdims=True)
    var = h.var(-1, keepdims=True)
    rep_actions = (h - mu) / jnp.sqrt(var + 1e-5) * params["ln_w"] + params["ln_b"]
    rep_actions = rep_actions @ params["out_a_W"] + params["out_a_b"]
    ra = rep_actions.reshape(_G, npart_per, 2)
    m = ra.max(1, keepdims=True)
    ex = jnp.exp(ra - m)
    den = ex.sum(1, keepdims=True)
    soft = ex / (den + 1e-16)
    actions = soft.transpose(0, 2, 1).reshape(_G, -1)
    rep = jnp.concatenate([
        _aggr(xd["part"], _G), _aggr(xd["torque"], _G), _aggr(xd["force"], _G),
    ], axis=1)
    repV = jax.nn.gelu(rep @ params["innet_W"] + params["innet_b"], approximate=False)
    repV = jax.nn.gelu(repV @ params["full_W"] + params["full_b"], approximate=False)
    V = jnp.tanh(repV @ params["outnet_W"] + params["outnet_b"])
    return actions, V


# speedup vs baseline: 15.5823x; 2.9344x over previous
import functools

import jax
import jax.numpy as jnp
from jax import lax
from jax.experimental import pallas as pl
from jax.experimental.pallas import tpu as pltpu
from jax.experimental.pallas import tpu_sc as plsc

_EDGE_TYPES = [("part", "pp", "part"), ("part", "pt", "torque"), ("torque", "tp", "part"),
               ("part", "pf", "force"), ("force", "fp", "part")]
_G = 50
_HID = 64
_BUCKET = 512
_CH = 512
_NW = 32


@functools.lru_cache(maxsize=None)
def _make_conv_kernel(E_pad, NB, N_src):
    NBK = (NB + _NW - 1) // _NW
    OFF_PAD = ((NB + 1 + 15) // 16) * 16
    mesh = plsc.VectorSubcoreMesh(core_axis_name="c", subcore_axis_name="s", num_cores=2)

    @functools.partial(
        pl.kernel, mesh=mesh,
        compiler_params=pltpu.CompilerParams(needs_layout_passes=False,
                                             use_tc_tiling_on_sc=False),
        out_type=jax.ShapeDtypeStruct((NB * _BUCKET * 64,), jnp.float32),
        scratch_types=[
            pltpu.VMEM((N_src,), jnp.float32),
            pltpu.VMEM((_BUCKET,), jnp.float32),
            pltpu.VMEM((_BUCKET,), jnp.float32),
            pltpu.VMEM((_BUCKET,), jnp.float32),
            pltpu.VMEM((_BUCKET * 64,), jnp.float32),
            pltpu.VMEM((_CH,), jnp.int32),
            pltpu.VMEM((_CH,), jnp.int32),
            pltpu.VMEM((_CH, 64), jnp.float32),
            pltpu.VMEM((OFF_PAD,), jnp.int32),
            pltpu.SemaphoreType.DMA,
        ],
    )
    def conv(s_hbm, d_hbm, offs_hbm, h_hbm, asrc_hbm, adst_hbm, out_hbm,
             asrc_v, adst_v, m_v, den_v, out_v, s_v, d_v, rows_v, offs_v, sem):
        wid = lax.axis_index("s") * 2 + lax.axis_index("c")
        pltpu.sync_copy(asrc_hbm, asrc_v)
        pltpu.sync_copy(offs_hbm, offs_v)
        iota = lax.iota(jnp.int32, 16)

        def read_off(b):
            goff = (b >> 4) << 4
            grp = offs_v[pl.ds(goff, 16)]
            lane = b & 15
            return jnp.max(jnp.where(iota == lane, grp, jnp.int32(-2147483648)))

        for k in range(NBK):
            b = wid + k * _NW

            @pl.when(b < NB)
            def _process():
                base = b * _BUCKET
                start = read_off(b)
                end = read_off(b + 1)
                c0 = start >> 9
                nch = (end - (c0 << 9) + _CH - 1) >> 9
                pltpu.sync_copy(adst_hbm.at[pl.ds(base, _BUCKET)], adst_v)

                def init_small(i, _):
                    m_v[pl.ds(i * 16, 16)] = jnp.full((16,), -1e30, jnp.float32)
                    den_v[pl.ds(i * 16, 16)] = jnp.zeros((16,), jnp.float32)
                    return 0
                lax.fori_loop(0, _BUCKET // 16, init_small, 0)

                def init_out(i, _):
                    out_v[pl.ds(i * 16, 16)] = jnp.zeros((16,), jnp.float32)
                    return 0
                lax.fori_loop(0, _BUCKET * 64 // 16, init_out, 0)

                def edge_vals(i):
                    s16 = s_v[pl.ds(i * 16, 16)]
                    d16 = d_v[pl.ds(i * 16, 16)]
                    dl = d16 - base
                    mask = (dl >= 0) & (dl < _BUCKET)
                    dl = jnp.clip(dl, 0, _BUCKET - 1)
                    a_s = plsc.load_gather(asrc_v, [s16], mask=mask)
                    a_d = plsc.load_gather(adst_v, [dl], mask=mask)
                    x = a_s + a_d
                    e = jnp.where(x >= 0, x, 0.2 * x)
                    return s16, dl, mask, e

                def sweep1(c, _):
                    ceoff = (c0 + c) * _CH
                    pltpu.sync_copy(s_hbm.at[pl.ds(ceoff, _CH)], s_v)
                    pltpu.sync_copy(d_hbm.at[pl.ds(ceoff, _CH)], d_v)

                    def grp1(i, _):
                        _, dl, mask, e = edge_vals(i)
                        e = jnp.where(mask, e, -1e30)
                        cur = plsc.load_gather(m_v, [dl])
                        plsc.store_scatter(m_v, [dl], jnp.maximum(cur, e))
                        return 0
                    lax.fori_loop(0, _CH // 16, grp1, 0)
                    return 0
                lax.fori_loop(0, nch, sweep1, 0)

                def sweep2(c, _):
                    ceoff = (c0 + c) * _CH
                    pltpu.sync_copy(s_hbm.at[pl.ds(ceoff, _CH)], s_v)
                    pltpu.sync_copy(d_hbm.at[pl.ds(ceoff, _CH)], d_v)

                    def grp2(i, _):
                        _, dl, mask, e = edge_vals(i)
                        mm = plsc.load_gather(m_v, [dl])
                        ex = jnp.where(mask, jnp.exp(e - mm), 0.0)
                        plsc.addupdate_scatter(den_v, [dl], ex)
                        return 0
                    lax.fori_loop(0, _CH // 16, grp2, 0)
                    return 0
                lax.fori_loop(0, nch, sweep2, 0)

                def invert(i, _):
                    den = den_v[pl.ds(i * 16, 16)]
                    den_v[pl.ds(i * 16, 16)] = 1.0 / (den + 1e-16)
                    return 0
                lax.fori_loop(0, _BUCKET // 16, invert, 0)

                def sweep3(c, _):
                    ceoff = (c0 + c) * _CH
                    pltpu.sync_copy(s_hbm.at[pl.ds(ceoff, _CH)], s_v)
                    pltpu.sync_copy(d_hbm.at[pl.ds(ceoff, _CH)], d_v)
                    pltpu.async_copy(h_hbm.at[s_v], rows_v, sem).wait()

                    def grp3(i, _):
                        _, dl, mask, e = edge_vals(i)
                        mm = plsc.load_gather(m_v, [dl])
                        inv = plsc.load_gather(den_v, [dl])
                        ex = jnp.where(mask, jnp.exp(e - mm), 0.0)
                        alpha = ex * inv
                        eidx = i * 16 + iota
                        dlb = dl * 64
                        for t in range(64):
                            tv = (iota + t) & 63
                            col = plsc.load_gather(rows_v, [eidx, tv])
                            plsc.addupdate_scatter(out_v, [dlb + tv], col * alpha)
                        return 0
                    lax.fori_loop(0, _CH // 16, grp3, 0)
                    return 0
                lax.fori_loop(0, nch, sweep3, 0)

                pltpu.sync_copy(out_v, out_hbm.at[pl.ds(b * (_BUCKET * 64), _BUCKET * 64)])

    return conv


def _sc_gat(h_src, a_src, a_dst, binned, num_dst):
    s_b, d_b, offs, E_pad = binned
    NB = (num_dst + _BUCKET - 1) // _BUCKET
    NBP = NB * _BUCKET
    N_src = h_src.shape[0]
    N_src_pad = ((N_src + 127) // 128) * 128
    a_src = jnp.pad(a_src, (0, N_src_pad - N_src))
    conv = _make_conv_kernel(E_pad, NB, N_src_pad)
    a_dst_p = jnp.pad(a_dst, (0, NBP - num_dst))
    out = conv(s_b, d_b, offs, h_src, a_src, a_dst_p)
    return out.reshape(NBP, 64)[:num_dst]


def _bin_edges(ei, num_dst):
    s, d = ei[0], ei[1]
    E = s.shape[0]
    NB = (num_dst + _BUCKET - 1) // _BUCKET
    OFF_PAD = ((NB + 1 + 15) // 16) * 16
    E_pad = E + 1024
    perm = jnp.argsort(d)
    s_b = jnp.pad(s[perm], (0, E_pad - E))
    d_b = jnp.pad(d[perm], (0, E_pad - E), constant_values=NB * _BUCKET + 1)
    offs = jnp.searchsorted(d_b[:E], jnp.arange(NB + 1, dtype=jnp.int32) * _BUCKET,
                            side="left").astype(jnp.int32)
    offs = jnp.pad(offs, (0, OFF_PAD - NB - 1), constant_values=E)
    return s_b, d_b, offs, E_pad


def _gat_dense(x_src, x_dst, p):
    h_src = x_src @ p["W_src"]
    a_src = h_src @ p["att_src"]
    a_dst = (x_dst @ p["W_dst"]) @ p["att_dst"]
    return h_src, a_src, a_dst


def _hetero_sc(xd, binned, lp, sizes, dst_types=("part", "torque", "force")):
    new = {}
    for (st, rel, dt) in _EDGE_TYPES:
        if dt not in dst_types:
            continue
        h_src, a_src, a_dst = _gat_dense(xd[st], xd[dt], lp[rel])
        o = _sc_gat(h_src, a_src, a_dst, binned[rel], sizes[dt]) + lp[rel]["bias"]
        new[dt] = o if dt not in new else new[dt] + o
    return new


def _aggr(x, ns):
    xg = x.reshape(_G, -1, x.shape[-1])
    return jnp.concatenate([xg.max(1), xg.min(1), xg.mean(1)], axis=1)


def kernel(mass, torque_x, force_x, params, part_state, edges, part_batch, part_id, torque_batch, force_batch):
    n_part = mass.shape[0]
    n_torque = torque_x.shape[0]
    n_force = force_x.shape[0]
    sizes = {"part": n_part, "torque": n_torque, "force": n_force}
    npart_per = n_part // _G

    binned = {rel: _bin_edges(edges[rel], sizes[dt]) for (st, rel, dt) in _EDGE_TYPES}

    state_idx = part_state[:, 0] + 2 * part_state[:, 1]
    reppart = jnp.concatenate([mass @ params["ep_W"], params["state_table"][state_idx]], axis=-1)
    xd = {"part": reppart, "torque": torque_x, "force": force_x}
    for li in range(4):
        xd = _hetero_sc(xd, binned, params["convs"][li], sizes)
        if li < 3:
            xd = {k: jax.nn.relu(v) for k, v in xd.items()}
    repA = _hetero_sc(xd, binned, params["actor"], sizes, dst_types=("part",))
    h = repA["part"]
    mu = h.mean(-1, keepdims=True)
    var = h.var(-1, keepdims=True)
    rep_actions = (h - mu) / jnp.sqrt(var + 1e-5) * params["ln_w"] + params["ln_b"]
    rep_actions = rep_actions @ params["out_a_W"] + params["out_a_b"]
    ra = rep_actions.reshape(_G, npart_per, 2)
    m = ra.max(1, keepdims=True)
    ex = jnp.exp(ra - m)
    den = ex.sum(1, keepdims=True)
    soft = ex / (den + 1e-16)
    actions = soft.transpose(0, 2, 1).reshape(_G, -1)
    rep = jnp.concatenate([
        _aggr(xd["part"], _G), _aggr(xd["torque"], _G), _aggr(xd["force"], _G),
    ], axis=1)
    repV = jax.nn.gelu(rep @ params["innet_W"] + params["innet_b"], approximate=False)
    repV = jax.nn.gelu(repV @ params["full_W"] + params["full_b"], approximate=False)
    V = jnp.tanh(repV @ params["outnet_W"] + params["outnet_b"])
    return actions, V


# speedup vs baseline: 18.1434x; 1.1644x over previous
import functools

import jax
import jax.numpy as jnp
from jax import lax
from jax.experimental import pallas as pl
from jax.experimental.pallas import tpu as pltpu
from jax.experimental.pallas import tpu_sc as plsc

_EDGE_TYPES = [("part", "pp", "part"), ("part", "pt", "torque"), ("torque", "tp", "part"),
               ("part", "pf", "force"), ("force", "fp", "part")]
_G = 50
_HID = 64
_BUCKET = 512
_CH = 512
_NW = 32


@functools.lru_cache(maxsize=None)
def _make_conv_kernel(E_pad, NB, N_src):
    NBK = (NB + _NW - 1) // _NW
    OFF_PAD = ((NB + 1 + 15) // 16) * 16
    mesh = plsc.VectorSubcoreMesh(core_axis_name="c", subcore_axis_name="s", num_cores=2)

    @functools.partial(
        pl.kernel, mesh=mesh,
        compiler_params=pltpu.CompilerParams(needs_layout_passes=False,
                                             use_tc_tiling_on_sc=False),
        out_type=[jax.ShapeDtypeStruct((NB * _BUCKET * 64,), jnp.float32),
                  jax.ShapeDtypeStruct((NB * _BUCKET,), jnp.float32)],
        scratch_types=[
            pltpu.VMEM((N_src,), jnp.float32),
            pltpu.VMEM((_BUCKET,), jnp.float32),
            pltpu.VMEM((_BUCKET,), jnp.float32),
            pltpu.VMEM((_BUCKET * 64,), jnp.float32),
            pltpu.VMEM((_CH,), jnp.int32),
            pltpu.VMEM((_CH,), jnp.int32),
            pltpu.VMEM((_CH, 64), jnp.float32),
            pltpu.VMEM((OFF_PAD,), jnp.int32),
            pltpu.VMEM((16,), jnp.float32),
            pltpu.SemaphoreType.DMA,
        ],
    )
    def conv(s_hbm, d_hbm, offs_hbm, h_hbm, asrc_hbm, adst_hbm, mglob_hbm,
             out_hbm, den_hbm,
             asrc_v, adst_v, den_v, out_v, s_v, d_v, rows_v, offs_v, mg_v, sem):
        wid = lax.axis_index("s") * 2 + lax.axis_index("c")
        pltpu.sync_copy(asrc_hbm, asrc_v)
        pltpu.sync_copy(offs_hbm, offs_v)
        pltpu.sync_copy(mglob_hbm, mg_v)
        iota = lax.iota(jnp.int32, 16)
        Mv = mg_v[pl.ds(0, 16)]

        def read_off(b):
            goff = (b >> 4) << 4
            grp = offs_v[pl.ds(goff, 16)]
            lane = b & 15
            return jnp.max(jnp.where(iota == lane, grp, jnp.int32(-2147483648)))

        for k in range(NBK):
            b = wid + k * _NW

            @pl.when(b < NB)
            def _process():
                base = b * _BUCKET
                start = read_off(b)
                end = read_off(b + 1)
                c0 = start >> 9
                nch = (end - (c0 << 9) + _CH - 1) >> 9
                pltpu.sync_copy(adst_hbm.at[pl.ds(base, _BUCKET)], adst_v)

                def init_small(i, _):
                    den_v[pl.ds(i * 16, 16)] = jnp.zeros((16,), jnp.float32)
                    return 0
                lax.fori_loop(0, _BUCKET // 16, init_small, 0)

                def init_out(i, _):
                    out_v[pl.ds(i * 16, 16)] = jnp.zeros((16,), jnp.float32)
                    return 0
                lax.fori_loop(0, _BUCKET * 64 // 16, init_out, 0)

                def sweep(c, _):
                    ceoff = (c0 + c) * _CH
                    pltpu.sync_copy(s_hbm.at[pl.ds(ceoff, _CH)], s_v)
                    pltpu.sync_copy(d_hbm.at[pl.ds(ceoff, _CH)], d_v)
                    pltpu.async_copy(h_hbm.at[s_v], rows_v, sem).wait()

                    def grp(i, _):
                        s16 = s_v[pl.ds(i * 16, 16)]
                        d16 = d_v[pl.ds(i * 16, 16)]
                        dl = d16 - base
                        mask = (dl >= 0) & (dl < _BUCKET)
                        dl = jnp.clip(dl, 0, _BUCKET - 1)
                        a_s = plsc.load_gather(asrc_v, [s16], mask=mask)
                        a_d = plsc.load_gather(adst_v, [dl], mask=mask)
                        x = a_s + a_d
                        e = jnp.where(x >= 0, x, 0.2 * x)
                        xm = Mv + a_d
                        mm = jnp.where(xm >= 0, xm, 0.2 * xm)
                        ex = jnp.where(mask, jnp.exp(e - mm), 0.0)
                        plsc.addupdate_scatter(den_v, [dl], ex)
                        eidx = i * 16 + iota
                        dlb = dl * 64
                        for t in range(64):
                            tv = (iota + t) & 63
                            col = plsc.load_gather(rows_v, [eidx, tv])
                            plsc.addupdate_scatter(out_v, [dlb + tv], col * ex)
                        return 0
                    lax.fori_loop(0, _CH // 16, grp, 0)
                    return 0
                lax.fori_loop(0, nch, sweep, 0)

                pltpu.sync_copy(out_v, out_hbm.at[pl.ds(b * (_BUCKET * 64), _BUCKET * 64)])
                pltpu.sync_copy(den_v, den_hbm.at[pl.ds(b * _BUCKET, _BUCKET)])

    return conv


def _sc_gat(h_src, a_src, a_dst, binned, num_dst):
    s_b, d_b, offs, E_pad = binned
    NB = (num_dst + _BUCKET - 1) // _BUCKET
    NBP = NB * _BUCKET
    N_src = h_src.shape[0]
    N_src_pad = ((N_src + 127) // 128) * 128
    mglob = jnp.full((16,), jnp.max(a_src), jnp.float32)
    a_src = jnp.pad(a_src, (0, N_src_pad - N_src))
    conv = _make_conv_kernel(E_pad, NB, N_src_pad)
    a_dst_p = jnp.pad(a_dst, (0, NBP - num_dst))
    out, den = conv(s_b, d_b, offs, h_src, a_src, a_dst_p, mglob)
    den = den[:num_dst]
    inv = jnp.where(den > 0, 1.0 / (den + 1e-16), 0.0)
    return out.reshape(NBP, 64)[:num_dst] * inv[:, None]


def _bin_edges(ei, num_dst):
    s, d = ei[0], ei[1]
    E = s.shape[0]
    NB = (num_dst + _BUCKET - 1) // _BUCKET
    OFF_PAD = ((NB + 1 + 15) // 16) * 16
    E_pad = E + 1024
    perm = jnp.argsort(d)
    s_b = jnp.pad(s[perm], (0, E_pad - E))
    d_b = jnp.pad(d[perm], (0, E_pad - E), constant_values=NB * _BUCKET + 1)
    offs = jnp.searchsorted(d_b[:E], jnp.arange(NB + 1, dtype=jnp.int32) * _BUCKET,
                            side="left").astype(jnp.int32)
    offs = jnp.pad(offs, (0, OFF_PAD - NB - 1), constant_values=E)
    return s_b, d_b, offs, E_pad


def _gat_dense(x_src, x_dst, p):
    h_src = x_src @ p["W_src"]
    a_src = h_src @ p["att_src"]
    a_dst = (x_dst @ p["W_dst"]) @ p["att_dst"]
    return h_src, a_src, a_dst


def _hetero_sc(xd, binned, lp, sizes, dst_types=("part", "torque", "force")):
    new = {}
    for (st, rel, dt) in _EDGE_TYPES:
        if dt not in dst_types:
            continue
        h_src, a_src, a_dst = _gat_dense(xd[st], xd[dt], lp[rel])
        o = _sc_gat(h_src, a_src, a_dst, binned[rel], sizes[dt]) + lp[rel]["bias"]
        new[dt] = o if dt not in new else new[dt] + o
    return new


def _aggr(x, ns):
    xg = x.reshape(_G, -1, x.shape[-1])
    return jnp.concatenate([xg.max(1), xg.min(1), xg.mean(1)], axis=1)


def kernel(mass, torque_x, force_x, params, part_state, edges, part_batch, part_id, torque_batch, force_batch):
    n_part = mass.shape[0]
    n_torque = torque_x.shape[0]
    n_force = force_x.shape[0]
    sizes = {"part": n_part, "torque": n_torque, "force": n_force}
    npart_per = n_part // _G

    binned = {rel: _bin_edges(edges[rel], sizes[dt]) for (st, rel, dt) in _EDGE_TYPES}

    state_idx = part_state[:, 0] + 2 * part_state[:, 1]
    reppart = jnp.concatenate([mass @ params["ep_W"], params["state_table"][state_idx]], axis=-1)
    xd = {"part": reppart, "torque": torque_x, "force": force_x}
    for li in range(4):
        xd = _hetero_sc(xd, binned, params["convs"][li], sizes)
        if li < 3:
            xd = {k: jax.nn.relu(v) for k, v in xd.items()}
    repA = _hetero_sc(xd, binned, params["actor"], sizes, dst_types=("part",))
    h = repA["part"]
    mu = h.mean(-1, keepdims=True)
    var = h.var(-1, keepdims=True)
    rep_actions = (h - mu) / jnp.sqrt(var + 1e-5) * params["ln_w"] + params["ln_b"]
    rep_actions = rep_actions @ params["out_a_W"] + params["out_a_b"]
    ra = rep_actions.reshape(_G, npart_per, 2)
    m = ra.max(1, keepdims=True)
    ex = jnp.exp(ra - m)
    den = ex.sum(1, keepdims=True)
    soft = ex / (den + 1e-16)
    actions = soft.transpose(0, 2, 1).reshape(_G, -1)
    rep = jnp.concatenate([
        _aggr(xd["part"], _G), _aggr(xd["torque"], _G), _aggr(xd["force"], _G),
    ], axis=1)
    repV = jax.nn.gelu(rep @ params["innet_W"] + params["innet_b"], approximate=False)
    repV = jax.nn.gelu(repV @ params["full_W"] + params["full_b"], approximate=False)
    V = jnp.tanh(repV @ params["outnet_W"] + params["outnet_b"])
    return actions, V
